# Initial kernel scaffold; baseline (speedup 1.0000x reference)
import jax
import jax.numpy as jnp
from jax.experimental import pallas as pl


def kernel(nodes, features, neigh_idx, W_enc, W_cls):
    raise NotImplementedError("write your pallas kernel here")


# speedup vs baseline: 3.5121x; 3.5121x over previous
import functools

import jax
import jax.numpy as jnp
from jax import lax
from jax.experimental import pallas as pl
from jax.experimental.pallas import tpu as pltpu
from jax.experimental.pallas import tpu_sc as plsc

N_NODES = 10000
D = 128
S = 32
C = 16
B = 10000
NW = 32
BP = 10240
PER_TILE = BP // NW
IDX_CHUNK = 80


def _sc_gather_mean():
    mesh = plsc.VectorSubcoreMesh(core_axis_name="c", subcore_axis_name="s")

    @functools.partial(
        pl.kernel,
        out_type=(
            jax.ShapeDtypeStruct((BP, D), jnp.float32),
            jax.ShapeDtypeStruct((BP, D), jnp.float32),
        ),
        mesh=mesh,
        scratch_types=(
            pltpu.VMEM((PER_TILE,), jnp.int32),
            pltpu.VMEM((PER_TILE, D), jnp.int32),
            pltpu.VMEM((PER_TILE, D), jnp.float32),
            pltpu.VMEM((S, D), jnp.float32),
            pltpu.VMEM((S, D), jnp.float32),
            pltpu.VMEM((IDX_CHUNK, D), jnp.float32),
            pltpu.VMEM((IDX_CHUNK, D), jnp.float32),
            pltpu.SemaphoreType.DMA,
            pltpu.SemaphoreType.DMA,
            pltpu.SemaphoreType.DMA,
            pltpu.SemaphoreType.DMA,
        ),
    )
    def k(nodes_hbm, feats_hbm, neigh_hbm, self_out, nsum_out,
          nodes_v, nb_v, nsum_v, buf0, buf1, sbuf0, sbuf1,
          sem_nb, sem_s, sem0, sem1):
        wid = lax.axis_index("s") * 2 + lax.axis_index("c")
        base = wid * PER_TILE
        n_chunks = PER_TILE // IDX_CHUNK

        pltpu.sync_copy(nodes_hbm.at[pl.ds(base, PER_TILE)], nodes_v)

        for j in range(n_chunks):
            idx = nodes_v.at[pl.ds(j * IDX_CHUNK, IDX_CHUNK)]
            pltpu.async_copy(neigh_hbm.at[idx],
                             nb_v.at[pl.ds(j * IDX_CHUNK, IDX_CHUNK)], sem_nb)
        for j in range(n_chunks):
            pltpu.make_async_copy(
                neigh_hbm.at[nodes_v.at[pl.ds(j * IDX_CHUNK, IDX_CHUNK)]],
                nb_v.at[pl.ds(j * IDX_CHUNK, IDX_CHUNK)], sem_nb).wait()

        bufs = (buf0, buf1)
        sems = (sem0, sem1)

        pltpu.async_copy(feats_hbm.at[nb_v.at[0, pl.ds(0, S)]], buf0, sem0)
        pltpu.async_copy(feats_hbm.at[nb_v.at[1, pl.ds(0, S)]], buf1, sem1)

        @pl.loop(0, PER_TILE, step=2)
        def _(i0):
            for b in range(2):
                i = i0 + b
                buf = bufs[b]
                sem = sems[b]
                pltpu.make_async_copy(
                    feats_hbm.at[nb_v.at[i, pl.ds(0, S)]], buf, sem).wait()
                for c in range(D // 16):
                    sl = pl.ds(c * 16, 16)
                    vals = [buf[s, sl] for s in range(S)]
                    while len(vals) > 1:
                        vals = [vals[t] + vals[t + 1]
                                for t in range(0, len(vals) - 1, 2)] + (
                                    [vals[-1]] if len(vals) % 2 else [])
                    nsum_v[i, sl] = vals[0]

                @pl.when(i < PER_TILE - 2)
                def _():
                    pltpu.async_copy(
                        feats_hbm.at[nb_v.at[i + 2, pl.ds(0, S)]], buf, sem)

        sbufs = (sbuf0, sbuf1)
        pltpu.async_copy(feats_hbm.at[nodes_v.at[pl.ds(0, IDX_CHUNK)]],
                         sbuf0, sem_s)
        for j in range(n_chunks):
            idx = nodes_v.at[pl.ds(j * IDX_CHUNK, IDX_CHUNK)]
            sb = sbufs[j % 2]
            pltpu.make_async_copy(feats_hbm.at[idx], sb, sem_s).wait()
            if j + 1 < n_chunks:
                nidx = nodes_v.at[pl.ds((j + 1) * IDX_CHUNK, IDX_CHUNK)]
                pltpu.async_copy(feats_hbm.at[nidx], sbufs[(j + 1) % 2], sem_s)
            pltpu.sync_copy(sb, self_out.at[pl.ds(base + j * IDX_CHUNK,
                                                  IDX_CHUNK)])

        pltpu.sync_copy(nsum_v, nsum_out.at[pl.ds(base, PER_TILE)])

    return k


TC_BLK = 1024


def _tc_dense(xs, xn, ws_t, wn_t, wc_t):
    def body(xs_ref, xn_ref, ws_ref, wn_ref, wc_ref, out_ref):
        h = jnp.dot(xs_ref[...], ws_ref[...], preferred_element_type=jnp.float32)
        h += jnp.dot(xn_ref[...], wn_ref[...], preferred_element_type=jnp.float32)
        h = jnp.maximum(h, 0.0)
        out_ref[...] = jnp.dot(h, wc_ref[...], preferred_element_type=jnp.float32)

    grid = BP // TC_BLK
    return pl.pallas_call(
        body,
        grid=(grid,),
        in_specs=[
            pl.BlockSpec((TC_BLK, D), lambda i: (i, 0)),
            pl.BlockSpec((TC_BLK, D), lambda i: (i, 0)),
            pl.BlockSpec((D, D), lambda i: (0, 0)),
            pl.BlockSpec((D, D), lambda i: (0, 0)),
            pl.BlockSpec((D, C), lambda i: (0, 0)),
        ],
        out_specs=pl.BlockSpec((TC_BLK, C), lambda i: (i, 0)),
        out_shape=jax.ShapeDtypeStruct((BP, C), jnp.float32),
    )(xs, xn, ws_t, wn_t, wc_t)


def kernel(nodes, features, neigh_idx, W_enc, W_cls):
    nodes_p = jnp.pad(nodes.astype(jnp.int32), (0, BP - B))
    neigh_p = jnp.pad(neigh_idx, ((0, 0), (0, D - S)))
    self_f, nsum = _sc_gather_mean()(nodes_p, features, neigh_p)
    ws_t = W_enc[:, :D].T
    wn_t = W_enc[:, D:].T * (1.0 / S)
    wc_t = W_cls.T
    scores = _tc_dense(self_f, nsum, ws_t, wn_t, wc_t)
    return scores[:B]
